# Initial kernel scaffold; baseline (speedup 1.0000x reference)
import jax
import jax.numpy as jnp
from jax.experimental import pallas as pl


def kernel(x, edge_index, Win, bin_, Wm0, bm0, wih0, whh0, bih0, bhh0, Wm1, bm1, wih1, whh1, bih1, bhh1, Wm2, bm2, wih2, whh2, bih2, bhh2, Wmu, bmu, Wls, bls, Wd1, bd1, Wd2, bd2):
    raise NotImplementedError("write your pallas kernel here")


# speedup vs baseline: 6.7147x; 6.7147x over previous
import functools

import jax
import jax.numpy as jnp
from jax import lax
from jax.experimental import pallas as pl
from jax.experimental.pallas import tpu as pltpu
from jax.experimental.pallas import tpu_sc as plsc

N = 2048
E = 65536
S = 32
L = 16

NC = 2
NS = 16
CH = 128
NBLK = NC * NS
NCHUNK = E // NBLK // CH

F32 = jnp.float32


def _enc0_body(xp_ref, winp_ref, bin_ref, wm_ref, bm_ref, state_ref, msg_ref):
    st = jnp.dot(xp_ref[...], winp_ref[...], preferred_element_type=F32)
    st = jnp.maximum(st + bin_ref[...], 0.0)
    state_ref[...] = st
    mg = jnp.dot(st, wm_ref[...], preferred_element_type=F32)
    msg_ref[...] = jnp.maximum(mg + bm_ref[...], 0.0)


_enc0 = pl.pallas_call(
    _enc0_body,
    out_shape=(
        jax.ShapeDtypeStruct((N, S), F32),
        jax.ShapeDtypeStruct((N, S), F32),
    ),
)


def _gru_core(h, agg_ref, w_refs, b_refs):
    a = agg_ref[0] + agg_ref[1]
    wri, wzi, wni, wrh, wzh, wnh = w_refs
    bri, bzi, bni, brh, bzh, bnh = b_refs
    gr = jnp.dot(a, wri[...], preferred_element_type=F32) + bri[...] \
        + jnp.dot(h, wrh[...], preferred_element_type=F32) + brh[...]
    gz = jnp.dot(a, wzi[...], preferred_element_type=F32) + bzi[...] \
        + jnp.dot(h, wzh[...], preferred_element_type=F32) + bzh[...]
    r = jax.nn.sigmoid(gr)
    z = jax.nn.sigmoid(gz)
    gn = jnp.dot(a, wni[...], preferred_element_type=F32) + bni[...] \
        + r * (jnp.dot(h, wnh[...], preferred_element_type=F32) + bnh[...])
    n = jnp.tanh(gn)
    return h + (1.0 - z) * n + z * h


def _gru_body(state_ref, agg_ref, wri, wzi, wni, wrh, wzh, wnh,
              bri, bzi, bni, brh, bzh, bnh, wm_ref, bm_ref,
              newstate_ref, msg_ref):
    hn = _gru_core(state_ref[...], agg_ref,
                   (wri, wzi, wni, wrh, wzh, wnh),
                   (bri, bzi, bni, brh, bzh, bnh))
    newstate_ref[...] = hn
    mg = jnp.dot(hn, wm_ref[...], preferred_element_type=F32)
    msg_ref[...] = jnp.maximum(mg + bm_ref[...], 0.0)


_gru_step = pl.pallas_call(
    _gru_body,
    out_shape=(
        jax.ShapeDtypeStruct((N, S), F32),
        jax.ShapeDtypeStruct((N, S), F32),
    ),
)


def _final_body(state_ref, agg_ref, wri, wzi, wni, wrh, wzh, wnh,
                bri, bzi, bni, brh, bzh, bnh,
                wmu_ref, bmu_ref, wls_ref, bls_ref,
                w1a_ref, w1b_ref, bd1_ref, bd1c_ref,
                mu_ref, ls_ref, ab_ref, bv_ref, abt_ref, bvt_ref):
    hn = _gru_core(state_ref[...], agg_ref,
                   (wri, wzi, wni, wrh, wzh, wnh),
                   (bri, bzi, bni, brh, bzh, bnh))
    mu = jnp.dot(hn, wmu_ref[...], preferred_element_type=F32) + bmu_ref[...]
    mu_ref[...] = mu
    ls_ref[...] = jnp.dot(hn, wls_ref[...], preferred_element_type=F32) + bls_ref[...]
    ab_ref[...] = jnp.dot(mu, w1a_ref[...], preferred_element_type=F32) + bd1_ref[...]
    bv_ref[...] = jnp.dot(mu, w1b_ref[...], preferred_element_type=F32)
    dn = (((0,), (1,)), ((), ()))
    abt_ref[...] = lax.dot_general(w1a_ref[...], mu, dn,
                                   preferred_element_type=F32) + bd1c_ref[...]
    bvt_ref[...] = lax.dot_general(w1b_ref[...], mu, dn,
                                   preferred_element_type=F32)


_final = pl.pallas_call(
    _final_body,
    out_shape=(
        jax.ShapeDtypeStruct((N, L), F32),
        jax.ShapeDtypeStruct((N, L), F32),
        jax.ShapeDtypeStruct((N, L), F32),
        jax.ShapeDtypeStruct((N, L), F32),
        jax.ShapeDtypeStruct((L, N), F32),
        jax.ShapeDtypeStruct((L, N), F32),
    ),
)


@functools.cache
def _make_sc_aggregate():
    mesh = plsc.VectorSubcoreMesh(core_axis_name="c", subcore_axis_name="s")

    @functools.partial(
        pl.kernel,
        mesh=mesh,
        out_type=jax.ShapeDtypeStruct((NC, N, S), F32),
        scratch_types=[
            pltpu.VMEM((NCHUNK, CH), jnp.int32),
            pltpu.VMEM((NCHUNK, CH), jnp.int32),
            pltpu.VMEM((CH, S), F32),
            pltpu.VMEM_SHARED((N, S), F32),
            pltpu.SemaphoreType.DMA,
        ],
        compiler_params=pltpu.CompilerParams(use_tc_tiling_on_sc=False),
    )
    def sc_aggregate(msg_hbm, src_hbm, dst_hbm, zeros_hbm, out_hbm,
                     src_v, dst_v, rows_v, acc_sh, sem):
        c = lax.axis_index("c")
        s = lax.axis_index("s")
        blk = c * NS + s
        pltpu.sync_copy(zeros_hbm, acc_sh.at[pl.ds(s * (N // NS), N // NS)])
        pltpu.sync_copy(src_hbm.at[blk], src_v)
        pltpu.sync_copy(dst_hbm.at[blk], dst_v)
        plsc.subcore_barrier()
        for j in range(NCHUNK):
            pltpu.async_copy(msg_hbm.at[src_v.at[j]], rows_v, sem).wait()
            pltpu.sync_copy(rows_v, acc_sh.at[dst_v.at[j]], add=True)
        plsc.subcore_barrier()
        pltpu.sync_copy(acc_sh.at[pl.ds(s * (N // NS), N // NS)],
                        out_hbm.at[c].at[pl.ds(s * (N // NS), N // NS)])

    return sc_aggregate


TI = 256
TJ = 256


def _dec_body(ab_ref, bv_ref, abt_ref, bvt_ref, w2_ref, bd2_ref, out_ref):
    a = ab_ref[...]
    b = bv_ref[...]
    at = abt_ref[...]
    bt = bvt_ref[...]
    acc = jnp.zeros((TI, TJ), F32)
    for k in range(L):
        wk = w2_ref[0, k]
        acc += wk * jnp.maximum(a[:, k:k + 1] + bt[k:k + 1, :], 0.0)
        acc += wk * jnp.maximum(b[:, k:k + 1] + at[k:k + 1, :], 0.0)
    out_ref[...] = jax.nn.sigmoid(0.5 * acc + bd2_ref[0, 0])


_decode = pl.pallas_call(
    _dec_body,
    grid=(N // TI, N // TJ),
    in_specs=[
        pl.BlockSpec((TI, L), lambda i, j: (i, 0)),
        pl.BlockSpec((TI, L), lambda i, j: (i, 0)),
        pl.BlockSpec((L, TJ), lambda i, j: (0, j)),
        pl.BlockSpec((L, TJ), lambda i, j: (0, j)),
        pl.BlockSpec(memory_space=pltpu.SMEM),
        pl.BlockSpec(memory_space=pltpu.SMEM),
    ],
    out_specs=pl.BlockSpec((TI, TJ), lambda i, j: (i, j)),
    out_shape=jax.ShapeDtypeStruct((N, N), F32),
)


def kernel(x, edge_index, Win, bin_, Wm0, bm0, wih0, whh0, bih0, bhh0,
           Wm1, bm1, wih1, whh1, bih1, bhh1,
           Wm2, bm2, wih2, whh2, bih2, bhh2,
           Wmu, bmu, Wls, bls, Wd1, bd1, Wd2, bd2):
    xp = jnp.pad(x, ((0, 0), (0, 1)))
    winp = jnp.pad(Win, ((0, 1), (0, 0)))
    row = lambda v: v.reshape(1, -1)
    src = edge_index[0].reshape(NBLK, NCHUNK, CH)
    dst = edge_index[1].reshape(NBLK, NCHUNK, CH)
    zeros = jnp.zeros((N // NS, S), F32)

    def gru_w(wih, whh, bih, bhh):
        ws = (wih[:S].T, wih[S:2 * S].T, wih[2 * S:].T,
              whh[:S].T, whh[S:2 * S].T, whh[2 * S:].T)
        bs = (row(bih[:S]), row(bih[S:2 * S]), row(bih[2 * S:]),
              row(bhh[:S]), row(bhh[S:2 * S]), row(bhh[2 * S:]))
        return ws + bs

    g0 = gru_w(wih0, whh0, bih0, bhh0)
    g1 = gru_w(wih1, whh1, bih1, bhh1)
    g2 = gru_w(wih2, whh2, bih2, bhh2)
    w1a, w1b = Wd1[:L], Wd1[L:]

    sc_aggregate = _make_sc_aggregate()
    state, msg = _enc0(xp, winp, row(bin_), Wm0, row(bm0))
    agg = sc_aggregate(msg, src, dst, zeros)
    state, msg = _gru_step(state, agg, *g0, Wm1, row(bm1))
    agg = sc_aggregate(msg, src, dst, zeros)
    state, msg = _gru_step(state, agg, *g1, Wm2, row(bm2))
    agg = sc_aggregate(msg, src, dst, zeros)
    mu, logstd, ab, bv, abt, bvt = _final(
        state, agg, *g2, Wmu, row(bmu), Wls, row(bls),
        w1a, w1b, row(bd1), bd1.reshape(L, 1))

    adj = _decode(ab, bv, abt, bvt, Wd2.reshape(1, L), bd2.reshape(1, 1))
    return (adj, mu, logstd)


# speedup vs baseline: 7.7130x; 1.1487x over previous
import functools

import jax
import jax.numpy as jnp
from jax import lax
from jax.experimental import pallas as pl
from jax.experimental.pallas import tpu as pltpu
from jax.experimental.pallas import tpu_sc as plsc

N = 2048
E = 65536
S = 32
L = 16

NC = 2
NS = 16
CH = 128
NBLK = NC * NS
NCHUNK = E // NBLK // CH

F32 = jnp.float32


def _enc0_body(xp_ref, winp_ref, bin_ref, wm_ref, bm_ref, state_ref, msg_ref):
    st = jnp.dot(xp_ref[...], winp_ref[...], preferred_element_type=F32)
    st = jnp.maximum(st + bin_ref[...], 0.0)
    state_ref[...] = st
    mg = jnp.dot(st, wm_ref[...], preferred_element_type=F32)
    msg_ref[...] = jnp.maximum(mg + bm_ref[...], 0.0)


_enc0 = pl.pallas_call(
    _enc0_body,
    out_shape=(
        jax.ShapeDtypeStruct((N, S), F32),
        jax.ShapeDtypeStruct((N, S), F32),
    ),
)


def _gru_core(h, agg_ref, w_refs, b_refs):
    a = agg_ref[0] + agg_ref[1]
    wri, wzi, wni, wrh, wzh, wnh = w_refs
    bri, bzi, bni, brh, bzh, bnh = b_refs
    gr = jnp.dot(a, wri[...], preferred_element_type=F32) + bri[...] \
        + jnp.dot(h, wrh[...], preferred_element_type=F32) + brh[...]
    gz = jnp.dot(a, wzi[...], preferred_element_type=F32) + bzi[...] \
        + jnp.dot(h, wzh[...], preferred_element_type=F32) + bzh[...]
    r = jax.nn.sigmoid(gr)
    z = jax.nn.sigmoid(gz)
    gn = jnp.dot(a, wni[...], preferred_element_type=F32) + bni[...] \
        + r * (jnp.dot(h, wnh[...], preferred_element_type=F32) + bnh[...])
    n = jnp.tanh(gn)
    return h + (1.0 - z) * n + z * h


def _gru_body(state_ref, agg_ref, wri, wzi, wni, wrh, wzh, wnh,
              bri, bzi, bni, brh, bzh, bnh, wm_ref, bm_ref,
              newstate_ref, msg_ref):
    hn = _gru_core(state_ref[...], agg_ref,
                   (wri, wzi, wni, wrh, wzh, wnh),
                   (bri, bzi, bni, brh, bzh, bnh))
    newstate_ref[...] = hn
    mg = jnp.dot(hn, wm_ref[...], preferred_element_type=F32)
    msg_ref[...] = jnp.maximum(mg + bm_ref[...], 0.0)


_gru_step = pl.pallas_call(
    _gru_body,
    out_shape=(
        jax.ShapeDtypeStruct((N, S), F32),
        jax.ShapeDtypeStruct((N, S), F32),
    ),
)


def _final_body(state_ref, agg_ref, wri, wzi, wni, wrh, wzh, wnh,
                bri, bzi, bni, brh, bzh, bnh,
                wmu_ref, bmu_ref, wls_ref, bls_ref,
                w1a_ref, w1b_ref, bd1_ref, bd1c_ref,
                mu_ref, ls_ref, ab_ref, bv_ref, abt_ref, bvt_ref):
    hn = _gru_core(state_ref[...], agg_ref,
                   (wri, wzi, wni, wrh, wzh, wnh),
                   (bri, bzi, bni, brh, bzh, bnh))
    mu = jnp.dot(hn, wmu_ref[...], preferred_element_type=F32) + bmu_ref[...]
    mu_ref[...] = mu
    ls_ref[...] = jnp.dot(hn, wls_ref[...], preferred_element_type=F32) + bls_ref[...]
    ab_ref[...] = jnp.dot(mu, w1a_ref[...], preferred_element_type=F32) + bd1_ref[...]
    bv_ref[...] = jnp.dot(mu, w1b_ref[...], preferred_element_type=F32)
    dn = (((0,), (1,)), ((), ()))
    abt_ref[...] = lax.dot_general(w1a_ref[...], mu, dn,
                                   preferred_element_type=F32) + bd1c_ref[...]
    bvt_ref[...] = lax.dot_general(w1b_ref[...], mu, dn,
                                   preferred_element_type=F32)


_final = pl.pallas_call(
    _final_body,
    out_shape=(
        jax.ShapeDtypeStruct((N, L), F32),
        jax.ShapeDtypeStruct((N, L), F32),
        jax.ShapeDtypeStruct((N, L), F32),
        jax.ShapeDtypeStruct((N, L), F32),
        jax.ShapeDtypeStruct((L, N), F32),
        jax.ShapeDtypeStruct((L, N), F32),
    ),
)


@functools.cache
def _make_sc_aggregate():
    mesh = plsc.VectorSubcoreMesh(core_axis_name="c", subcore_axis_name="s")

    @functools.partial(
        pl.kernel,
        mesh=mesh,
        out_type=jax.ShapeDtypeStruct((NC, N, S), F32),
        scratch_types=[
            pltpu.VMEM((NCHUNK, CH), jnp.int32),
            pltpu.VMEM((NCHUNK, CH), jnp.int32),
            pltpu.VMEM((2, CH, S), F32),
            pltpu.VMEM_SHARED((N, S), F32),
            pltpu.SemaphoreType.DMA,
            pltpu.SemaphoreType.DMA,
        ],
        compiler_params=pltpu.CompilerParams(use_tc_tiling_on_sc=False),
    )
    def sc_aggregate(msg_hbm, src_hbm, dst_hbm, zeros_hbm, out_hbm,
                     src_v, dst_v, rows_v, acc_sh, sem_a, sem_b):
        c = lax.axis_index("c")
        s = lax.axis_index("s")
        blk = c * NS + s
        pltpu.sync_copy(zeros_hbm, acc_sh.at[pl.ds(s * (N // NS), N // NS)])
        pltpu.sync_copy(src_hbm.at[blk], src_v)
        pltpu.sync_copy(dst_hbm.at[blk], dst_v)
        plsc.subcore_barrier()
        sems = (sem_a, sem_b)
        handles = [None, None]
        handles[0] = pltpu.async_copy(msg_hbm.at[src_v.at[0]], rows_v.at[0],
                                      sems[0])
        for j in range(NCHUNK):
            if j + 1 < NCHUNK:
                handles[(j + 1) % 2] = pltpu.async_copy(
                    msg_hbm.at[src_v.at[j + 1]], rows_v.at[(j + 1) % 2],
                    sems[(j + 1) % 2])
            handles[j % 2].wait()
            pltpu.sync_copy(rows_v.at[j % 2], acc_sh.at[dst_v.at[j]], add=True)
        plsc.subcore_barrier()
        pltpu.sync_copy(acc_sh.at[pl.ds(s * (N // NS), N // NS)],
                        out_hbm.at[c].at[pl.ds(s * (N // NS), N // NS)])

    return sc_aggregate


TI = 256
TJ = 2048


def _dec_body(ab_ref, bv_ref, abt_ref, bvt_ref, w2_ref, bd2_ref, out_ref):
    a = ab_ref[...]
    b = bv_ref[...]
    at = abt_ref[...]
    bt = bvt_ref[...]
    acc = jnp.zeros((TI, TJ), F32)
    for k in range(L):
        wk = w2_ref[0, k]
        t = jnp.maximum(a[:, k:k + 1] + bt[k:k + 1, :], 0.0) \
            + jnp.maximum(b[:, k:k + 1] + at[k:k + 1, :], 0.0)
        acc += wk * t
    out_ref[...] = jax.nn.sigmoid(0.5 * acc + bd2_ref[0, 0])


_decode = pl.pallas_call(
    _dec_body,
    grid=(N // TI, N // TJ),
    in_specs=[
        pl.BlockSpec((TI, L), lambda i, j: (i, 0)),
        pl.BlockSpec((TI, L), lambda i, j: (i, 0)),
        pl.BlockSpec((L, TJ), lambda i, j: (0, j)),
        pl.BlockSpec((L, TJ), lambda i, j: (0, j)),
        pl.BlockSpec(memory_space=pltpu.SMEM),
        pl.BlockSpec(memory_space=pltpu.SMEM),
    ],
    out_specs=pl.BlockSpec((TI, TJ), lambda i, j: (i, j)),
    out_shape=jax.ShapeDtypeStruct((N, N), F32),
)


def kernel(x, edge_index, Win, bin_, Wm0, bm0, wih0, whh0, bih0, bhh0,
           Wm1, bm1, wih1, whh1, bih1, bhh1,
           Wm2, bm2, wih2, whh2, bih2, bhh2,
           Wmu, bmu, Wls, bls, Wd1, bd1, Wd2, bd2):
    xp = jnp.pad(x, ((0, 0), (0, 1)))
    winp = jnp.pad(Win, ((0, 1), (0, 0)))
    row = lambda v: v.reshape(1, -1)
    src = edge_index[0].reshape(NBLK, NCHUNK, CH)
    dst = edge_index[1].reshape(NBLK, NCHUNK, CH)
    zeros = jnp.zeros((N // NS, S), F32)

    def gru_w(wih, whh, bih, bhh):
        ws = (wih[:S].T, wih[S:2 * S].T, wih[2 * S:].T,
              whh[:S].T, whh[S:2 * S].T, whh[2 * S:].T)
        bs = (row(bih[:S]), row(bih[S:2 * S]), row(bih[2 * S:]),
              row(bhh[:S]), row(bhh[S:2 * S]), row(bhh[2 * S:]))
        return ws + bs

    g0 = gru_w(wih0, whh0, bih0, bhh0)
    g1 = gru_w(wih1, whh1, bih1, bhh1)
    g2 = gru_w(wih2, whh2, bih2, bhh2)
    w1a, w1b = Wd1[:L], Wd1[L:]

    sc_aggregate = _make_sc_aggregate()
    state, msg = _enc0(xp, winp, row(bin_), Wm0, row(bm0))
    agg = sc_aggregate(msg, src, dst, zeros)
    state, msg = _gru_step(state, agg, *g0, Wm1, row(bm1))
    agg = sc_aggregate(msg, src, dst, zeros)
    state, msg = _gru_step(state, agg, *g1, Wm2, row(bm2))
    agg = sc_aggregate(msg, src, dst, zeros)
    mu, logstd, ab, bv, abt, bvt = _final(
        state, agg, *g2, Wmu, row(bmu), Wls, row(bls),
        w1a, w1b, row(bd1), bd1.reshape(L, 1))

    adj = _decode(ab, bv, abt, bvt, Wd2.reshape(1, L), bd2.reshape(1, 1))
    return (adj, mu, logstd)


# speedup vs baseline: 8.2232x; 1.0662x over previous
import functools

import jax
import jax.numpy as jnp
from jax import lax
from jax.experimental import pallas as pl
from jax.experimental.pallas import tpu as pltpu
from jax.experimental.pallas import tpu_sc as plsc

N = 2048
E = 65536
S = 32
L = 16

NC = 2
NS = 16
CH = 128
NBLK = NC * NS
NCHUNK = E // NBLK // CH

P = 4
PR = N // P
PW = P * S

F32 = jnp.float32


def _enc0_body(xp_ref, winp_ref, bin_ref, wm_ref, bm_ref, state_ref, msg_ref):
    st = jnp.dot(xp_ref[...], winp_ref[...], preferred_element_type=F32)
    st = jnp.maximum(st + bin_ref[...], 0.0)
    state_ref[...] = st
    mg = jnp.dot(st, wm_ref[...], preferred_element_type=F32)
    msg_ref[...] = jnp.maximum(mg + bm_ref[...], 0.0)


_enc0 = pl.pallas_call(
    _enc0_body,
    out_shape=(
        jax.ShapeDtypeStruct((PR, PW), F32),
        jax.ShapeDtypeStruct((PR, PW), F32),
    ),
)


def _gru_core(h, agg_ref, w_refs, b_refs):
    a = agg_ref[0] + agg_ref[1]
    wri, wzi, wni, wrh, wzh, wnh = w_refs
    bri, bzi, bni, brh, bzh, bnh = b_refs
    gr = jnp.dot(a, wri[...], preferred_element_type=F32) + bri[...] \
        + jnp.dot(h, wrh[...], preferred_element_type=F32) + brh[...]
    gz = jnp.dot(a, wzi[...], preferred_element_type=F32) + bzi[...] \
        + jnp.dot(h, wzh[...], preferred_element_type=F32) + bzh[...]
    r = jax.nn.sigmoid(gr)
    z = jax.nn.sigmoid(gz)
    gn = jnp.dot(a, wni[...], preferred_element_type=F32) + bni[...] \
        + r * (jnp.dot(h, wnh[...], preferred_element_type=F32) + bnh[...])
    n = jnp.tanh(gn)
    return h + (1.0 - z) * n + z * h


def _gru_body(state_ref, agg_ref, wri, wzi, wni, wrh, wzh, wnh,
              bri, bzi, bni, brh, bzh, bnh, wm_ref, bm_ref,
              newstate_ref, msg_ref):
    hn = _gru_core(state_ref[...], agg_ref,
                   (wri, wzi, wni, wrh, wzh, wnh),
                   (bri, bzi, bni, brh, bzh, bnh))
    newstate_ref[...] = hn
    mg = jnp.dot(hn, wm_ref[...], preferred_element_type=F32)
    msg_ref[...] = jnp.maximum(mg + bm_ref[...], 0.0)


_gru_step = pl.pallas_call(
    _gru_body,
    out_shape=(
        jax.ShapeDtypeStruct((PR, PW), F32),
        jax.ShapeDtypeStruct((PR, PW), F32),
    ),
)


def _final_body(state_ref, agg_ref, wri, wzi, wni, wrh, wzh, wnh,
                bri, bzi, bni, brh, bzh, bnh,
                wmu_ref, bmu_ref, wls_ref, bls_ref,
                w1a_ref, w1b_ref, bd1_ref, bd1c_ref,
                mu_ref, ls_ref, ab_ref, bv_ref, abt_ref, bvt_ref):
    hn_p = _gru_core(state_ref[...], agg_ref,
                     (wri, wzi, wni, wrh, wzh, wnh),
                     (bri, bzi, bni, brh, bzh, bnh))
    hn = jnp.concatenate([hn_p[:, u * S:(u + 1) * S] for u in range(P)], axis=0)
    mu = jnp.dot(hn, wmu_ref[...], preferred_element_type=F32) + bmu_ref[...]
    mu_ref[...] = mu
    ls_ref[...] = jnp.dot(hn, wls_ref[...], preferred_element_type=F32) + bls_ref[...]
    ab_ref[...] = jnp.dot(mu, w1a_ref[...], preferred_element_type=F32) + bd1_ref[...]
    bv_ref[...] = jnp.dot(mu, w1b_ref[...], preferred_element_type=F32)
    dn = (((0,), (1,)), ((), ()))
    abt_ref[...] = lax.dot_general(w1a_ref[...], mu, dn,
                                   preferred_element_type=F32) + bd1c_ref[...]
    bvt_ref[...] = lax.dot_general(w1b_ref[...], mu, dn,
                                   preferred_element_type=F32)


_final = pl.pallas_call(
    _final_body,
    out_shape=(
        jax.ShapeDtypeStruct((N, L), F32),
        jax.ShapeDtypeStruct((N, L), F32),
        jax.ShapeDtypeStruct((N, L), F32),
        jax.ShapeDtypeStruct((N, L), F32),
        jax.ShapeDtypeStruct((L, N), F32),
        jax.ShapeDtypeStruct((L, N), F32),
    ),
)


@functools.cache
def _make_sc_aggregate():
    mesh = plsc.VectorSubcoreMesh(core_axis_name="c", subcore_axis_name="s")

    @functools.partial(
        pl.kernel,
        mesh=mesh,
        out_type=jax.ShapeDtypeStruct((NC, N, S), F32),
        scratch_types=[
            pltpu.VMEM((NCHUNK, CH), jnp.int32),
            pltpu.VMEM((NCHUNK, CH), jnp.int32),
            pltpu.VMEM((2, CH, S), F32),
            pltpu.VMEM_SHARED((N, S), F32),
            pltpu.SemaphoreType.DMA,
            pltpu.SemaphoreType.DMA,
        ],
        compiler_params=pltpu.CompilerParams(use_tc_tiling_on_sc=False),
    )
    def sc_aggregate(msg_hbm, src_hbm, dst_hbm, zeros_hbm, out_hbm,
                     src_v, dst_v, rows_v, acc_sh, sem_a, sem_b):
        c = lax.axis_index("c")
        s = lax.axis_index("s")
        blk = c * NS + s
        pltpu.sync_copy(zeros_hbm, acc_sh.at[pl.ds(s * (N // NS), N // NS)])
        pltpu.sync_copy(src_hbm.at[blk], src_v)
        pltpu.sync_copy(dst_hbm.at[blk], dst_v)
        plsc.subcore_barrier()
        sems = (sem_a, sem_b)
        handles = [None, None]
        handles[0] = pltpu.async_copy(msg_hbm.at[src_v.at[0]], rows_v.at[0],
                                      sems[0])
        for j in range(NCHUNK):
            if j + 1 < NCHUNK:
                handles[(j + 1) % 2] = pltpu.async_copy(
                    msg_hbm.at[src_v.at[j + 1]], rows_v.at[(j + 1) % 2],
                    sems[(j + 1) % 2])
            handles[j % 2].wait()
            pltpu.sync_copy(rows_v.at[j % 2], acc_sh.at[dst_v.at[j]], add=True)
        plsc.subcore_barrier()
        pltpu.sync_copy(acc_sh.at[pl.ds(s * (N // NS), N // NS)],
                        out_hbm.at[c].at[pl.ds(s * (N // NS), N // NS)])

    return sc_aggregate


TI = 256
TJ = 2048


def _dec_body(ab_ref, bv_ref, abt_ref, bvt_ref, w2_ref, bd2_ref, out_ref):
    a = ab_ref[...]
    b = bv_ref[...]
    at = abt_ref[...]
    bt = bvt_ref[...]
    acc = jnp.zeros((TI, TJ), F32)
    for k in range(L):
        wk = w2_ref[0, k]
        t = jnp.maximum(a[:, k:k + 1] + bt[k:k + 1, :], 0.0) \
            + jnp.maximum(b[:, k:k + 1] + at[k:k + 1, :], 0.0)
        acc += wk * t
    out_ref[...] = jax.nn.sigmoid(0.5 * acc + bd2_ref[0, 0])


_decode = pl.pallas_call(
    _dec_body,
    grid=(N // TI, N // TJ),
    in_specs=[
        pl.BlockSpec((TI, L), lambda i, j: (i, 0)),
        pl.BlockSpec((TI, L), lambda i, j: (i, 0)),
        pl.BlockSpec((L, TJ), lambda i, j: (0, j)),
        pl.BlockSpec((L, TJ), lambda i, j: (0, j)),
        pl.BlockSpec(memory_space=pltpu.SMEM),
        pl.BlockSpec(memory_space=pltpu.SMEM),
    ],
    out_specs=pl.BlockSpec((TI, TJ), lambda i, j: (i, j)),
    out_shape=jax.ShapeDtypeStruct((N, N), F32),
)


def kernel(x, edge_index, Win, bin_, Wm0, bm0, wih0, whh0, bih0, bhh0,
           Wm1, bm1, wih1, whh1, bih1, bhh1,
           Wm2, bm2, wih2, whh2, bih2, bhh2,
           Wmu, bmu, Wls, bls, Wd1, bd1, Wd2, bd2):
    eye4 = jnp.eye(P, dtype=F32)
    bd = lambda w: jnp.kron(eye4, w)
    row = lambda v: jnp.tile(v, P).reshape(1, PW)
    xpp = jnp.pad(x, ((0, 0), (0, 1))).reshape(P, PR, 8) \
        .transpose(1, 0, 2).reshape(PR, P * 8)
    winp = bd(jnp.pad(Win, ((0, 1), (0, 0))))
    perm = lambda idx: (idx % PR) * P + idx // PR
    src = perm(edge_index[0]).reshape(NBLK, NCHUNK, CH)
    dst = perm(edge_index[1]).reshape(NBLK, NCHUNK, CH)
    zeros = jnp.zeros((N // NS, S), F32)

    def gru_w(wih, whh, bih, bhh):
        ws = (bd(wih[:S].T), bd(wih[S:2 * S].T), bd(wih[2 * S:].T),
              bd(whh[:S].T), bd(whh[S:2 * S].T), bd(whh[2 * S:].T))
        bs = (row(bih[:S]), row(bih[S:2 * S]), row(bih[2 * S:]),
              row(bhh[:S]), row(bhh[S:2 * S]), row(bhh[2 * S:]))
        return ws + bs

    g0 = gru_w(wih0, whh0, bih0, bhh0)
    g1 = gru_w(wih1, whh1, bih1, bhh1)
    g2 = gru_w(wih2, whh2, bih2, bhh2)
    w1a, w1b = Wd1[:L], Wd1[L:]
    rw = lambda v: v.reshape(1, -1)

    sc_aggregate = _make_sc_aggregate()
    state, msg = _enc0(xpp, winp, row(bin_), bd(Wm0), row(bm0))
    agg = sc_aggregate(msg.reshape(N, S), src, dst, zeros)
    state, msg = _gru_step(state, agg.reshape(NC, PR, PW), *g0, bd(Wm1), row(bm1))
    agg = sc_aggregate(msg.reshape(N, S), src, dst, zeros)
    state, msg = _gru_step(state, agg.reshape(NC, PR, PW), *g1, bd(Wm2), row(bm2))
    agg = sc_aggregate(msg.reshape(N, S), src, dst, zeros)
    mu, logstd, ab, bv, abt, bvt = _final(
        state, agg.reshape(NC, PR, PW), *g2, Wmu, rw(bmu), Wls, rw(bls),
        w1a, w1b, rw(bd1), bd1.reshape(L, 1))

    adj = _decode(ab, bv, abt, bvt, Wd2.reshape(1, L), bd2.reshape(1, 1))
    return (adj, mu, logstd)


# speedup vs baseline: 8.4412x; 1.0265x over previous
import functools

import jax
import jax.numpy as jnp
from jax import lax
from jax.experimental import pallas as pl
from jax.experimental.pallas import tpu as pltpu
from jax.experimental.pallas import tpu_sc as plsc

N = 2048
E = 65536
S = 32
L = 16

NC = 2
NS = 16
CH = 128
NBLK = NC * NS
NCHUNK = E // NBLK // CH

P = 4
PR = N // P
PW = P * S

F32 = jnp.float32


_DNT = (((1,), (1,)), ((), ()))


def _pmul(a, w, b, u_width=S):
    parts = [jnp.dot(a[:, u * u_width:(u + 1) * u_width], w,
                     preferred_element_type=F32) + b
             for u in range(P)]
    return jnp.concatenate(parts, axis=1)


def _enc0_body(xp_ref, winp_ref, bin_ref, wm_ref, bm_ref, state_ref, msg_ref):
    st = jnp.maximum(_pmul(xp_ref[...], winp_ref[...], bin_ref[...], 8), 0.0)
    state_ref[...] = st
    msg_ref[...] = jnp.maximum(_pmul(st, wm_ref[...], bm_ref[...]), 0.0)


_enc0 = pl.pallas_call(
    _enc0_body,
    out_shape=(
        jax.ShapeDtypeStruct((PR, PW), F32),
        jax.ShapeDtypeStruct((PR, PW), F32),
    ),
)


def _half_gate(x, w, b):
    parts = [lax.dot_general(x[:, u * S:(u + 1) * S], w, _DNT,
                             preferred_element_type=F32) + b
             for u in range(P)]
    return jnp.concatenate(parts, axis=1)


def _gru_core(h, agg_ref, wih_ref, whh_ref, bih_ref, bhh_ref):
    a = agg_ref[0] + agg_ref[1]
    wih, whh = wih_ref[...], whh_ref[...]
    bih, bhh = bih_ref[...], bhh_ref[...]
    blk = lambda w, g: w[g * S:(g + 1) * S, :]
    bb = lambda b, g: b[:, g * S:(g + 1) * S]
    r = jax.nn.sigmoid(_half_gate(a, blk(wih, 0), bb(bih, 0))
                       + _half_gate(h, blk(whh, 0), bb(bhh, 0)))
    z = jax.nn.sigmoid(_half_gate(a, blk(wih, 1), bb(bih, 1))
                       + _half_gate(h, blk(whh, 1), bb(bhh, 1)))
    n = jnp.tanh(_half_gate(a, blk(wih, 2), bb(bih, 2))
                 + r * _half_gate(h, blk(whh, 2), bb(bhh, 2)))
    return h + (1.0 - z) * n + z * h


def _gru_body(state_ref, agg_ref, wih_ref, whh_ref, bih_ref, bhh_ref,
              wm_ref, bm_ref, newstate_ref, msg_ref):
    hn = _gru_core(state_ref[...], agg_ref, wih_ref, whh_ref,
                   bih_ref, bhh_ref)
    newstate_ref[...] = hn
    msg_ref[...] = jnp.maximum(_pmul(hn, wm_ref[...], bm_ref[...]), 0.0)


_gru_step = pl.pallas_call(
    _gru_body,
    out_shape=(
        jax.ShapeDtypeStruct((PR, PW), F32),
        jax.ShapeDtypeStruct((PR, PW), F32),
    ),
)


def _final_body(state_ref, agg_ref, wih_ref, whh_ref, bih_ref, bhh_ref,
                wmu_ref, bmu_ref, wls_ref, bls_ref,
                w1a_ref, w1b_ref, bd1_ref, bd1c_ref,
                mu_ref, ls_ref, ab_ref, bv_ref, abt_ref, bvt_ref):
    hn_p = _gru_core(state_ref[...], agg_ref, wih_ref, whh_ref,
                     bih_ref, bhh_ref)
    hn = jnp.concatenate([hn_p[:, u * S:(u + 1) * S] for u in range(P)], axis=0)
    mu = jnp.dot(hn, wmu_ref[...], preferred_element_type=F32) + bmu_ref[...]
    mu_ref[...] = mu
    ls_ref[...] = jnp.dot(hn, wls_ref[...], preferred_element_type=F32) + bls_ref[...]
    ab_ref[...] = jnp.dot(mu, w1a_ref[...], preferred_element_type=F32) + bd1_ref[...]
    bv_ref[...] = jnp.dot(mu, w1b_ref[...], preferred_element_type=F32)
    dn = (((0,), (1,)), ((), ()))
    abt_ref[...] = lax.dot_general(w1a_ref[...], mu, dn,
                                   preferred_element_type=F32) + bd1c_ref[...]
    bvt_ref[...] = lax.dot_general(w1b_ref[...], mu, dn,
                                   preferred_element_type=F32)


_final = pl.pallas_call(
    _final_body,
    out_shape=(
        jax.ShapeDtypeStruct((N, L), F32),
        jax.ShapeDtypeStruct((N, L), F32),
        jax.ShapeDtypeStruct((N, L), F32),
        jax.ShapeDtypeStruct((N, L), F32),
        jax.ShapeDtypeStruct((L, N), F32),
        jax.ShapeDtypeStruct((L, N), F32),
    ),
)


@functools.cache
def _make_sc_aggregate():
    mesh = plsc.VectorSubcoreMesh(core_axis_name="c", subcore_axis_name="s")

    @functools.partial(
        pl.kernel,
        mesh=mesh,
        out_type=jax.ShapeDtypeStruct((NC, N, S), F32),
        scratch_types=[
            pltpu.VMEM((NCHUNK, CH), jnp.int32),
            pltpu.VMEM((NCHUNK, CH), jnp.int32),
            pltpu.VMEM((2, CH, S), F32),
            pltpu.VMEM_SHARED((N, S), F32),
            pltpu.SemaphoreType.DMA,
            pltpu.SemaphoreType.DMA,
        ],
        compiler_params=pltpu.CompilerParams(use_tc_tiling_on_sc=False),
    )
    def sc_aggregate(msg_hbm, src_hbm, dst_hbm, zeros_hbm, out_hbm,
                     src_v, dst_v, rows_v, acc_sh, sem_a, sem_b):
        c = lax.axis_index("c")
        s = lax.axis_index("s")
        blk = c * NS + s
        pltpu.sync_copy(zeros_hbm, acc_sh.at[pl.ds(s * (N // NS), N // NS)])
        pltpu.sync_copy(src_hbm.at[blk], src_v)
        pltpu.sync_copy(dst_hbm.at[blk], dst_v)
        plsc.subcore_barrier()
        sems = (sem_a, sem_b)
        handles = [None, None]
        handles[0] = pltpu.async_copy(msg_hbm.at[src_v.at[0]], rows_v.at[0],
                                      sems[0])
        for j in range(NCHUNK):
            if j + 1 < NCHUNK:
                handles[(j + 1) % 2] = pltpu.async_copy(
                    msg_hbm.at[src_v.at[j + 1]], rows_v.at[(j + 1) % 2],
                    sems[(j + 1) % 2])
            handles[j % 2].wait()
            pltpu.sync_copy(rows_v.at[j % 2], acc_sh.at[dst_v.at[j]], add=True)
        plsc.subcore_barrier()
        pltpu.sync_copy(acc_sh.at[pl.ds(s * (N // NS), N // NS)],
                        out_hbm.at[c].at[pl.ds(s * (N // NS), N // NS)])

    return sc_aggregate


TI = 256
TJ = 2048


def _dec_body(ab_ref, bv_ref, abt_ref, bvt_ref, w2_ref, bd2_ref, out_ref):
    a = ab_ref[...]
    b = bv_ref[...]
    at = abt_ref[...]
    bt = bvt_ref[...]
    acc = jnp.zeros((TI, TJ), F32)
    for k in range(L):
        wk = w2_ref[0, k]
        t = jnp.maximum(a[:, k:k + 1] + bt[k:k + 1, :], 0.0) \
            + jnp.maximum(b[:, k:k + 1] + at[k:k + 1, :], 0.0)
        acc += wk * t
    out_ref[...] = jax.nn.sigmoid(0.5 * acc + bd2_ref[0, 0])


_decode = pl.pallas_call(
    _dec_body,
    grid=(N // TI, N // TJ),
    in_specs=[
        pl.BlockSpec((TI, L), lambda i, j: (i, 0)),
        pl.BlockSpec((TI, L), lambda i, j: (i, 0)),
        pl.BlockSpec((L, TJ), lambda i, j: (0, j)),
        pl.BlockSpec((L, TJ), lambda i, j: (0, j)),
        pl.BlockSpec(memory_space=pltpu.SMEM),
        pl.BlockSpec(memory_space=pltpu.SMEM),
    ],
    out_specs=pl.BlockSpec((TI, TJ), lambda i, j: (i, j)),
    out_shape=jax.ShapeDtypeStruct((N, N), F32),
)


def kernel(x, edge_index, Win, bin_, Wm0, bm0, wih0, whh0, bih0, bhh0,
           Wm1, bm1, wih1, whh1, bih1, bhh1,
           Wm2, bm2, wih2, whh2, bih2, bhh2,
           Wmu, bmu, Wls, bls, Wd1, bd1, Wd2, bd2):
    xpp = jnp.pad(x, ((0, 0), (0, 1))).reshape(P, PR, 8) \
        .transpose(1, 0, 2).reshape(PR, P * 8)
    winp = jnp.pad(Win, ((0, 1), (0, 0)))
    perm = lambda idx: (idx % PR) * P + idx // PR
    src = perm(edge_index[0]).reshape(NBLK, NCHUNK, CH)
    dst = perm(edge_index[1]).reshape(NBLK, NCHUNK, CH)
    zeros = jnp.zeros((N // NS, S), F32)
    w1a, w1b = Wd1[:L], Wd1[L:]
    rw = lambda v: v.reshape(1, -1)

    sc_aggregate = _make_sc_aggregate()
    state, msg = _enc0(xpp, winp, rw(bin_), Wm0, rw(bm0))
    agg = sc_aggregate(msg.reshape(N, S), src, dst, zeros)
    state, msg = _gru_step(state, agg.reshape(NC, PR, PW),
                           wih0, whh0, rw(bih0), rw(bhh0), Wm1, rw(bm1))
    agg = sc_aggregate(msg.reshape(N, S), src, dst, zeros)
    state, msg = _gru_step(state, agg.reshape(NC, PR, PW),
                           wih1, whh1, rw(bih1), rw(bhh1), Wm2, rw(bm2))
    agg = sc_aggregate(msg.reshape(N, S), src, dst, zeros)
    mu, logstd, ab, bv, abt, bvt = _final(
        state, agg.reshape(NC, PR, PW),
        wih2, whh2, rw(bih2), rw(bhh2), Wmu, rw(bmu), Wls, rw(bls),
        w1a, w1b, rw(bd1), bd1.reshape(L, 1))

    adj = _decode(ab, bv, abt, bvt, Wd2.reshape(1, L), bd2.reshape(1, 1))
    return (adj, mu, logstd)


# speedup vs baseline: 8.8320x; 1.0463x over previous
import functools

import jax
import jax.numpy as jnp
from jax import lax
from jax.experimental import pallas as pl
from jax.experimental.pallas import tpu as pltpu
from jax.experimental.pallas import tpu_sc as plsc

N = 2048
E = 65536
S = 32
L = 16

NC = 2
NS = 16
CH = 128
NBLK = NC * NS
NCHUNK = E // NBLK // CH

P = 4
PR = N // P
PW = P * S

F32 = jnp.float32


_DNT = (((1,), (1,)), ((), ()))


def _pmul(a, w, b, u_width=S):
    parts = [jnp.dot(a[:, u * u_width:(u + 1) * u_width], w,
                     preferred_element_type=F32) + b
             for u in range(P)]
    return jnp.concatenate(parts, axis=1)


def _enc0_body(xp_ref, winp_ref, bin_ref, wm_ref, bm_ref, state_ref, msg_ref):
    st = jnp.maximum(_pmul(xp_ref[...], winp_ref[...], bin_ref[...], 8), 0.0)
    state_ref[...] = st
    msg_ref[...] = jnp.maximum(_pmul(st, wm_ref[...], bm_ref[...]), 0.0)


_enc0 = pl.pallas_call(
    _enc0_body,
    out_shape=(
        jax.ShapeDtypeStruct((PR, PW), F32),
        jax.ShapeDtypeStruct((PR, PW), F32),
    ),
)


def _half_gate(x, w, b):
    parts = [lax.dot_general(x[:, u * S:(u + 1) * S], w, _DNT,
                             preferred_element_type=F32) + b
             for u in range(P)]
    return jnp.concatenate(parts, axis=1)


def _gru_core(h, agg_ref, wih_ref, whh_ref, bih_ref, bhh_ref):
    a = agg_ref[0] + agg_ref[1]
    wih, whh = wih_ref[...], whh_ref[...]
    bih, bhh = bih_ref[...], bhh_ref[...]
    blk = lambda w, g: w[g * S:(g + 1) * S, :]
    bb = lambda b, g: b[:, g * S:(g + 1) * S]
    r = jax.nn.sigmoid(_half_gate(a, blk(wih, 0), bb(bih, 0))
                       + _half_gate(h, blk(whh, 0), bb(bhh, 0)))
    z = jax.nn.sigmoid(_half_gate(a, blk(wih, 1), bb(bih, 1))
                       + _half_gate(h, blk(whh, 1), bb(bhh, 1)))
    n = jnp.tanh(_half_gate(a, blk(wih, 2), bb(bih, 2))
                 + r * _half_gate(h, blk(whh, 2), bb(bhh, 2)))
    return h + (1.0 - z) * n + z * h


def _gru_body(state_ref, agg_ref, wih_ref, whh_ref, bih_ref, bhh_ref,
              wm_ref, bm_ref, newstate_ref, msg_ref):
    hn = _gru_core(state_ref[...], agg_ref, wih_ref, whh_ref,
                   bih_ref, bhh_ref)
    newstate_ref[...] = hn
    msg_ref[...] = jnp.maximum(_pmul(hn, wm_ref[...], bm_ref[...]), 0.0)


_gru_step = pl.pallas_call(
    _gru_body,
    out_shape=(
        jax.ShapeDtypeStruct((PR, PW), F32),
        jax.ShapeDtypeStruct((PR, PW), F32),
    ),
)


def _final_body(state_ref, agg_ref, wih_ref, whh_ref, bih_ref, bhh_ref,
                wmu_ref, bmu_ref, wls_ref, bls_ref,
                w1a_ref, w1b_ref, bd1_ref, bd1c_ref,
                mu_ref, ls_ref, ab_ref, bv_ref, abt_ref, bvt_ref):
    hn_p = _gru_core(state_ref[...], agg_ref, wih_ref, whh_ref,
                     bih_ref, bhh_ref)
    hn = jnp.concatenate([hn_p[:, u * S:(u + 1) * S] for u in range(P)], axis=0)
    mu = jnp.dot(hn, wmu_ref[...], preferred_element_type=F32) + bmu_ref[...]
    mu_ref[...] = mu
    ls_ref[...] = jnp.dot(hn, wls_ref[...], preferred_element_type=F32) + bls_ref[...]
    ab_ref[...] = jnp.dot(mu, w1a_ref[...], preferred_element_type=F32) + bd1_ref[...]
    bv_ref[...] = jnp.dot(mu, w1b_ref[...], preferred_element_type=F32)
    dn = (((0,), (1,)), ((), ()))
    abt_ref[...] = lax.dot_general(w1a_ref[...], mu, dn,
                                   preferred_element_type=F32) + bd1c_ref[...]
    bvt_ref[...] = lax.dot_general(w1b_ref[...], mu, dn,
                                   preferred_element_type=F32)


_final = pl.pallas_call(
    _final_body,
    out_shape=(
        jax.ShapeDtypeStruct((N, L), F32),
        jax.ShapeDtypeStruct((N, L), F32),
        jax.ShapeDtypeStruct((N, L), F32),
        jax.ShapeDtypeStruct((N, L), F32),
        jax.ShapeDtypeStruct((L, N), F32),
        jax.ShapeDtypeStruct((L, N), F32),
    ),
)


@functools.cache
def _make_sc_aggregate():
    mesh = plsc.VectorSubcoreMesh(core_axis_name="c", subcore_axis_name="s")

    @functools.partial(
        pl.kernel,
        mesh=mesh,
        out_type=jax.ShapeDtypeStruct((NC, N, S), F32),
        scratch_types=[
            pltpu.VMEM((NCHUNK, CH), jnp.int32),
            pltpu.VMEM((NCHUNK, CH), jnp.int32),
            pltpu.VMEM((2, CH, S), F32),
            pltpu.VMEM_SHARED((N, S), F32),
            pltpu.VMEM_SHARED((N, S), F32),
            pltpu.SemaphoreType.DMA,
            pltpu.SemaphoreType.DMA,
        ],
        compiler_params=pltpu.CompilerParams(use_tc_tiling_on_sc=False),
    )
    def sc_aggregate(msg_hbm, src_hbm, dst_hbm, zeros_hbm, out_hbm,
                     src_v, dst_v, rows_v, acc_sh, msg_sh, sem_a, sem_b):
        c = lax.axis_index("c")
        s = lax.axis_index("s")
        blk = c * NS + s
        rsl = pl.ds(s * (N // NS), N // NS)
        pltpu.sync_copy(zeros_hbm, acc_sh.at[rsl])
        pltpu.sync_copy(msg_hbm.at[rsl], msg_sh.at[rsl])
        pltpu.sync_copy(src_hbm.at[blk], src_v)
        pltpu.sync_copy(dst_hbm.at[blk], dst_v)
        plsc.subcore_barrier()
        sems = (sem_a, sem_b)
        handles = [None, None]
        handles[0] = pltpu.async_copy(msg_sh.at[src_v.at[0]], rows_v.at[0],
                                      sems[0])
        for j in range(NCHUNK):
            if j + 1 < NCHUNK:
                handles[(j + 1) % 2] = pltpu.async_copy(
                    msg_sh.at[src_v.at[j + 1]], rows_v.at[(j + 1) % 2],
                    sems[(j + 1) % 2])
            handles[j % 2].wait()
            pltpu.sync_copy(rows_v.at[j % 2], acc_sh.at[dst_v.at[j]], add=True)
        plsc.subcore_barrier()
        pltpu.sync_copy(acc_sh.at[pl.ds(s * (N // NS), N // NS)],
                        out_hbm.at[c].at[pl.ds(s * (N // NS), N // NS)])

    return sc_aggregate


TI = 256
TJ = 2048


def _dec_body(ab_ref, bv_ref, abt_ref, bvt_ref, w2_ref, bd2_ref, out_ref):
    a = ab_ref[...]
    b = bv_ref[...]
    at = abt_ref[...]
    bt = bvt_ref[...]
    acc = jnp.zeros((TI, TJ), F32)
    for k in range(L):
        wk = w2_ref[0, k]
        t = jnp.maximum(a[:, k:k + 1] + bt[k:k + 1, :], 0.0) \
            + jnp.maximum(b[:, k:k + 1] + at[k:k + 1, :], 0.0)
        acc += wk * t
    out_ref[...] = jax.nn.sigmoid(0.5 * acc + bd2_ref[0, 0])


_decode = pl.pallas_call(
    _dec_body,
    grid=(N // TI, N // TJ),
    in_specs=[
        pl.BlockSpec((TI, L), lambda i, j: (i, 0)),
        pl.BlockSpec((TI, L), lambda i, j: (i, 0)),
        pl.BlockSpec((L, TJ), lambda i, j: (0, j)),
        pl.BlockSpec((L, TJ), lambda i, j: (0, j)),
        pl.BlockSpec(memory_space=pltpu.SMEM),
        pl.BlockSpec(memory_space=pltpu.SMEM),
    ],
    out_specs=pl.BlockSpec((TI, TJ), lambda i, j: (i, j)),
    out_shape=jax.ShapeDtypeStruct((N, N), F32),
)


def kernel(x, edge_index, Win, bin_, Wm0, bm0, wih0, whh0, bih0, bhh0,
           Wm1, bm1, wih1, whh1, bih1, bhh1,
           Wm2, bm2, wih2, whh2, bih2, bhh2,
           Wmu, bmu, Wls, bls, Wd1, bd1, Wd2, bd2):
    xpp = jnp.pad(x, ((0, 0), (0, 1))).reshape(P, PR, 8) \
        .transpose(1, 0, 2).reshape(PR, P * 8)
    winp = jnp.pad(Win, ((0, 1), (0, 0)))
    perm = lambda idx: (idx % PR) * P + idx // PR
    src = perm(edge_index[0]).reshape(NBLK, NCHUNK, CH)
    dst = perm(edge_index[1]).reshape(NBLK, NCHUNK, CH)
    zeros = jnp.zeros((N // NS, S), F32)
    w1a, w1b = Wd1[:L], Wd1[L:]
    rw = lambda v: v.reshape(1, -1)

    sc_aggregate = _make_sc_aggregate()
    state, msg = _enc0(xpp, winp, rw(bin_), Wm0, rw(bm0))
    agg = sc_aggregate(msg.reshape(N, S), src, dst, zeros)
    state, msg = _gru_step(state, agg.reshape(NC, PR, PW),
                           wih0, whh0, rw(bih0), rw(bhh0), Wm1, rw(bm1))
    agg = sc_aggregate(msg.reshape(N, S), src, dst, zeros)
    state, msg = _gru_step(state, agg.reshape(NC, PR, PW),
                           wih1, whh1, rw(bih1), rw(bhh1), Wm2, rw(bm2))
    agg = sc_aggregate(msg.reshape(N, S), src, dst, zeros)
    mu, logstd, ab, bv, abt, bvt = _final(
        state, agg.reshape(NC, PR, PW),
        wih2, whh2, rw(bih2), rw(bhh2), Wmu, rw(bmu), Wls, rw(bls),
        w1a, w1b, rw(bd1), bd1.reshape(L, 1))

    adj = _decode(ab, bv, abt, bvt, Wd2.reshape(1, L), bd2.reshape(1, 1))
    return (adj, mu, logstd)


# speedup vs baseline: 11.1487x; 1.2623x over previous
import functools

import jax
import jax.numpy as jnp
from jax import lax
from jax.experimental import pallas as pl
from jax.experimental.pallas import tpu as pltpu
from jax.experimental.pallas import tpu_sc as plsc

N = 2048
E = 65536
S = 32
L = 16

NC = 2
NS = 16
CH = 128
NBLK = NC * NS
NCHUNK = E // NBLK // CH

P = 4
PR = N // P
PW = P * S

F32 = jnp.float32


_DNT = (((1,), (1,)), ((), ()))


def _pmul(a, w, b, u_width=S):
    parts = [jnp.dot(a[:, u * u_width:(u + 1) * u_width], w,
                     preferred_element_type=F32) + b
             for u in range(P)]
    return jnp.concatenate(parts, axis=1)


def _enc0_body(xp_ref, winp_ref, bin_ref, wm_ref, bm_ref, state_ref, msg_ref):
    st = jnp.maximum(_pmul(xp_ref[...], winp_ref[...], bin_ref[...], 8), 0.0)
    state_ref[...] = st
    msg_ref[...] = jnp.maximum(_pmul(st, wm_ref[...], bm_ref[...]), 0.0)


_enc0 = pl.pallas_call(
    _enc0_body,
    out_shape=(
        jax.ShapeDtypeStruct((PR, PW), F32),
        jax.ShapeDtypeStruct((PR, PW), F32),
    ),
)


def _half_gate(x, w, b):
    parts = [lax.dot_general(x[:, u * S:(u + 1) * S], w, _DNT,
                             preferred_element_type=F32) + b
             for u in range(P)]
    return jnp.concatenate(parts, axis=1)


def _gru_core(h, agg_ref, wih_ref, whh_ref, bih_ref, bhh_ref):
    a = agg_ref[0] + agg_ref[1]
    wih, whh = wih_ref[...], whh_ref[...]
    bih, bhh = bih_ref[...], bhh_ref[...]
    blk = lambda w, g: w[g * S:(g + 1) * S, :]
    bb = lambda b, g: b[:, g * S:(g + 1) * S]
    r = jax.nn.sigmoid(_half_gate(a, blk(wih, 0), bb(bih, 0))
                       + _half_gate(h, blk(whh, 0), bb(bhh, 0)))
    z = jax.nn.sigmoid(_half_gate(a, blk(wih, 1), bb(bih, 1))
                       + _half_gate(h, blk(whh, 1), bb(bhh, 1)))
    n = jnp.tanh(_half_gate(a, blk(wih, 2), bb(bih, 2))
                 + r * _half_gate(h, blk(whh, 2), bb(bhh, 2)))
    return h + (1.0 - z) * n + z * h


def _gru_body(state_ref, agg_ref, wih_ref, whh_ref, bih_ref, bhh_ref,
              wm_ref, bm_ref, newstate_ref, msg_ref):
    hn = _gru_core(state_ref[...], agg_ref, wih_ref, whh_ref,
                   bih_ref, bhh_ref)
    newstate_ref[...] = hn
    msg_ref[...] = jnp.maximum(_pmul(hn, wm_ref[...], bm_ref[...]), 0.0)


_gru_step = pl.pallas_call(
    _gru_body,
    out_shape=(
        jax.ShapeDtypeStruct((PR, PW), F32),
        jax.ShapeDtypeStruct((PR, PW), F32),
    ),
)


def _final_body(state_ref, agg_ref, wih_ref, whh_ref, bih_ref, bhh_ref,
                wmu_ref, bmu_ref, wls_ref, bls_ref,
                w1a_ref, w1b_ref, bd1_ref, bd1c_ref,
                mu_ref, ls_ref, ab_ref, bv_ref, abt_ref, bvt_ref):
    hn_p = _gru_core(state_ref[...], agg_ref, wih_ref, whh_ref,
                     bih_ref, bhh_ref)
    hn = jnp.concatenate([hn_p[:, u * S:(u + 1) * S] for u in range(P)], axis=0)
    mu = jnp.dot(hn, wmu_ref[...], preferred_element_type=F32) + bmu_ref[...]
    mu_ref[...] = mu
    ls_ref[...] = jnp.dot(hn, wls_ref[...], preferred_element_type=F32) + bls_ref[...]
    bf = jnp.bfloat16
    ab_ref[...] = (jnp.dot(mu, w1a_ref[...], preferred_element_type=F32)
                   + bd1_ref[...]).astype(bf)
    bv_ref[...] = jnp.dot(mu, w1b_ref[...],
                          preferred_element_type=F32).astype(bf)
    dn = (((0,), (1,)), ((), ()))
    abt_ref[...] = (lax.dot_general(w1a_ref[...], mu, dn,
                                    preferred_element_type=F32)
                    + bd1c_ref[...]).astype(bf)
    bvt_ref[...] = lax.dot_general(w1b_ref[...], mu, dn,
                                   preferred_element_type=F32).astype(bf)


_final = pl.pallas_call(
    _final_body,
    out_shape=(
        jax.ShapeDtypeStruct((N, L), F32),
        jax.ShapeDtypeStruct((N, L), F32),
        jax.ShapeDtypeStruct((N, L), jnp.bfloat16),
        jax.ShapeDtypeStruct((N, L), jnp.bfloat16),
        jax.ShapeDtypeStruct((L, N), jnp.bfloat16),
        jax.ShapeDtypeStruct((L, N), jnp.bfloat16),
    ),
)


@functools.cache
def _make_sc_aggregate():
    mesh = plsc.VectorSubcoreMesh(core_axis_name="c", subcore_axis_name="s")

    @functools.partial(
        pl.kernel,
        mesh=mesh,
        out_type=jax.ShapeDtypeStruct((NC, N, S), F32),
        scratch_types=[
            pltpu.VMEM((NCHUNK, CH), jnp.int32),
            pltpu.VMEM((NCHUNK, CH), jnp.int32),
            pltpu.VMEM((2, CH, S), F32),
            pltpu.VMEM_SHARED((N, S), F32),
            pltpu.VMEM_SHARED((N, S), F32),
            pltpu.SemaphoreType.DMA,
            pltpu.SemaphoreType.DMA,
        ],
        compiler_params=pltpu.CompilerParams(use_tc_tiling_on_sc=False),
    )
    def sc_aggregate(msg_hbm, src_hbm, dst_hbm, zeros_hbm, out_hbm,
                     src_v, dst_v, rows_v, acc_sh, msg_sh, sem_a, sem_b):
        c = lax.axis_index("c")
        s = lax.axis_index("s")
        blk = c * NS + s
        rsl = pl.ds(s * (N // NS), N // NS)
        pltpu.sync_copy(zeros_hbm, acc_sh.at[rsl])
        pltpu.sync_copy(msg_hbm.at[rsl], msg_sh.at[rsl])
        pltpu.sync_copy(src_hbm.at[blk], src_v)
        pltpu.sync_copy(dst_hbm.at[blk], dst_v)
        plsc.subcore_barrier()
        sems = (sem_a, sem_b)
        handles = [None, None]
        handles[0] = pltpu.async_copy(msg_sh.at[src_v.at[0]], rows_v.at[0],
                                      sems[0])
        for j in range(NCHUNK):
            if j + 1 < NCHUNK:
                handles[(j + 1) % 2] = pltpu.async_copy(
                    msg_sh.at[src_v.at[j + 1]], rows_v.at[(j + 1) % 2],
                    sems[(j + 1) % 2])
            handles[j % 2].wait()
            pltpu.sync_copy(rows_v.at[j % 2], acc_sh.at[dst_v.at[j]], add=True)
        plsc.subcore_barrier()
        pltpu.sync_copy(acc_sh.at[pl.ds(s * (N // NS), N // NS)],
                        out_hbm.at[c].at[pl.ds(s * (N // NS), N // NS)])

    return sc_aggregate


TI = 256
TJ = 2048


def _dec_body(ab_ref, bv_ref, abt_ref, bvt_ref, w2_ref, bd2_ref, out_ref):
    a = ab_ref[...]
    b = bv_ref[...]
    at = abt_ref[...]
    bt = bvt_ref[...]
    zero = jnp.bfloat16(0.0)
    acc = jnp.zeros((TI, TJ), jnp.bfloat16)
    for k in range(L):
        wk = w2_ref[0, k]
        t = jnp.maximum(a[:, k:k + 1] + bt[k:k + 1, :], zero) \
            + jnp.maximum(b[:, k:k + 1] + at[k:k + 1, :], zero)
        acc += wk * t
    out_ref[...] = jax.nn.sigmoid(0.5 * acc.astype(F32) + bd2_ref[0, 0])


_decode = pl.pallas_call(
    _dec_body,
    grid=(N // TI, N // TJ),
    in_specs=[
        pl.BlockSpec((TI, L), lambda i, j: (i, 0)),
        pl.BlockSpec((TI, L), lambda i, j: (i, 0)),
        pl.BlockSpec((L, TJ), lambda i, j: (0, j)),
        pl.BlockSpec((L, TJ), lambda i, j: (0, j)),
        pl.BlockSpec(memory_space=pltpu.SMEM),
        pl.BlockSpec(memory_space=pltpu.SMEM),
    ],
    out_specs=pl.BlockSpec((TI, TJ), lambda i, j: (i, j)),
    out_shape=jax.ShapeDtypeStruct((N, N), F32),
)


def kernel(x, edge_index, Win, bin_, Wm0, bm0, wih0, whh0, bih0, bhh0,
           Wm1, bm1, wih1, whh1, bih1, bhh1,
           Wm2, bm2, wih2, whh2, bih2, bhh2,
           Wmu, bmu, Wls, bls, Wd1, bd1, Wd2, bd2):
    xpp = jnp.pad(x, ((0, 0), (0, 1))).reshape(P, PR, 8) \
        .transpose(1, 0, 2).reshape(PR, P * 8)
    winp = jnp.pad(Win, ((0, 1), (0, 0)))
    perm = lambda idx: (idx % PR) * P + idx // PR
    src = perm(edge_index[0]).reshape(NBLK, NCHUNK, CH)
    dst = perm(edge_index[1]).reshape(NBLK, NCHUNK, CH)
    zeros = jnp.zeros((N // NS, S), F32)
    w1a, w1b = Wd1[:L], Wd1[L:]
    rw = lambda v: v.reshape(1, -1)

    sc_aggregate = _make_sc_aggregate()
    state, msg = _enc0(xpp, winp, rw(bin_), Wm0, rw(bm0))
    agg = sc_aggregate(msg.reshape(N, S), src, dst, zeros)
    state, msg = _gru_step(state, agg.reshape(NC, PR, PW),
                           wih0, whh0, rw(bih0), rw(bhh0), Wm1, rw(bm1))
    agg = sc_aggregate(msg.reshape(N, S), src, dst, zeros)
    state, msg = _gru_step(state, agg.reshape(NC, PR, PW),
                           wih1, whh1, rw(bih1), rw(bhh1), Wm2, rw(bm2))
    agg = sc_aggregate(msg.reshape(N, S), src, dst, zeros)
    mu, logstd, ab, bv, abt, bvt = _final(
        state, agg.reshape(NC, PR, PW),
        wih2, whh2, rw(bih2), rw(bhh2), Wmu, rw(bmu), Wls, rw(bls),
        w1a, w1b, rw(bd1), bd1.reshape(L, 1))

    adj = _decode(ab, bv, abt, bvt,
                  Wd2.reshape(1, L).astype(jnp.bfloat16), bd2.reshape(1, 1))
    return (adj, mu, logstd)


# speedup vs baseline: 11.1997x; 1.0046x over previous
import functools

import jax
import jax.numpy as jnp
from jax import lax
from jax.experimental import pallas as pl
from jax.experimental.pallas import tpu as pltpu
from jax.experimental.pallas import tpu_sc as plsc

N = 2048
E = 65536
S = 32
L = 16

NC = 2
NS = 16
CH = 128
NBLK = NC * NS
NCHUNK = E // NBLK // CH

P = 4
PR = N // P
PW = P * S

F32 = jnp.float32


_DNT = (((1,), (1,)), ((), ()))


def _pmul(a, w, b, u_width=S):
    parts = [jnp.dot(a[:, u * u_width:(u + 1) * u_width], w,
                     preferred_element_type=F32) + b
             for u in range(P)]
    return jnp.concatenate(parts, axis=1)


def _enc0_body(x_ref, win_ref, bin_ref, wm_ref, bm_ref, state_ref, msg_ref):
    win, b = win_ref[...], bin_ref[...]
    parts = [jnp.maximum(
        jnp.dot(x_ref[u * PR:(u + 1) * PR, :], win,
                preferred_element_type=F32) + b, 0.0) for u in range(P)]
    st = jnp.concatenate(parts, axis=1)
    state_ref[...] = st
    msg_ref[...] = jnp.maximum(_pmul(st, wm_ref[...], bm_ref[...]), 0.0)


_enc0 = pl.pallas_call(
    _enc0_body,
    out_shape=(
        jax.ShapeDtypeStruct((PR, PW), F32),
        jax.ShapeDtypeStruct((PR, PW), F32),
    ),
)


def _half_gate(x, w, b):
    parts = [lax.dot_general(x[:, u * S:(u + 1) * S], w, _DNT,
                             preferred_element_type=F32) + b
             for u in range(P)]
    return jnp.concatenate(parts, axis=1)


def _gru_core(h, agg_ref, wih_ref, whh_ref, bih_ref, bhh_ref):
    a = agg_ref[0] + agg_ref[1]
    wih, whh = wih_ref[...], whh_ref[...]
    bih, bhh = bih_ref[...], bhh_ref[...]
    blk = lambda w, g: w[g * S:(g + 1) * S, :]
    bb = lambda b, g: b[:, g * S:(g + 1) * S]
    r = jax.nn.sigmoid(_half_gate(a, blk(wih, 0), bb(bih, 0))
                       + _half_gate(h, blk(whh, 0), bb(bhh, 0)))
    z = jax.nn.sigmoid(_half_gate(a, blk(wih, 1), bb(bih, 1))
                       + _half_gate(h, blk(whh, 1), bb(bhh, 1)))
    n = jnp.tanh(_half_gate(a, blk(wih, 2), bb(bih, 2))
                 + r * _half_gate(h, blk(whh, 2), bb(bhh, 2)))
    return h + (1.0 - z) * n + z * h


def _gru_body(state_ref, agg_ref, wih_ref, whh_ref, bih_ref, bhh_ref,
              wm_ref, bm_ref, newstate_ref, msg_ref):
    hn = _gru_core(state_ref[...], agg_ref, wih_ref, whh_ref,
                   bih_ref, bhh_ref)
    newstate_ref[...] = hn
    msg_ref[...] = jnp.maximum(_pmul(hn, wm_ref[...], bm_ref[...]), 0.0)


_gru_step = pl.pallas_call(
    _gru_body,
    out_shape=(
        jax.ShapeDtypeStruct((PR, PW), F32),
        jax.ShapeDtypeStruct((PR, PW), F32),
    ),
)


def _final_body(state_ref, agg_ref, wih_ref, whh_ref, bih_ref, bhh_ref,
                wmu_ref, bmu_ref, wls_ref, bls_ref,
                w1a_ref, w1b_ref, bd1_ref, bd1c_ref,
                mu_ref, ls_ref, ab_ref, bv_ref, abt_ref, bvt_ref):
    hn_p = _gru_core(state_ref[...], agg_ref, wih_ref, whh_ref,
                     bih_ref, bhh_ref)
    hn = jnp.concatenate([hn_p[:, u * S:(u + 1) * S] for u in range(P)], axis=0)
    mu = jnp.dot(hn, wmu_ref[...], preferred_element_type=F32) + bmu_ref[...]
    mu_ref[...] = mu
    ls_ref[...] = jnp.dot(hn, wls_ref[...], preferred_element_type=F32) + bls_ref[...]
    bf = jnp.bfloat16
    ab_ref[...] = (jnp.dot(mu, w1a_ref[...], preferred_element_type=F32)
                   + bd1_ref[...]).astype(bf)
    bv_ref[...] = jnp.dot(mu, w1b_ref[...],
                          preferred_element_type=F32).astype(bf)
    dn = (((0,), (1,)), ((), ()))
    abt_ref[...] = (lax.dot_general(w1a_ref[...], mu, dn,
                                    preferred_element_type=F32)
                    + bd1c_ref[...]).astype(bf)
    bvt_ref[...] = lax.dot_general(w1b_ref[...], mu, dn,
                                   preferred_element_type=F32).astype(bf)


_final = pl.pallas_call(
    _final_body,
    out_shape=(
        jax.ShapeDtypeStruct((N, L), F32),
        jax.ShapeDtypeStruct((N, L), F32),
        jax.ShapeDtypeStruct((N, L), jnp.bfloat16),
        jax.ShapeDtypeStruct((N, L), jnp.bfloat16),
        jax.ShapeDtypeStruct((L, N), jnp.bfloat16),
        jax.ShapeDtypeStruct((L, N), jnp.bfloat16),
    ),
)


@functools.cache
def _make_sc_aggregate():
    mesh = plsc.VectorSubcoreMesh(core_axis_name="c", subcore_axis_name="s")

    @functools.partial(
        pl.kernel,
        mesh=mesh,
        out_type=jax.ShapeDtypeStruct((NC, N, S), F32),
        scratch_types=[
            pltpu.VMEM((NCHUNK, CH), jnp.int32),
            pltpu.VMEM((NCHUNK, CH), jnp.int32),
            pltpu.VMEM((2, CH, S), F32),
            pltpu.VMEM_SHARED((N, S), F32),
            pltpu.VMEM_SHARED((N, S), F32),
            pltpu.SemaphoreType.DMA,
            pltpu.SemaphoreType.DMA,
        ],
        compiler_params=pltpu.CompilerParams(use_tc_tiling_on_sc=False),
    )
    def sc_aggregate(msg_hbm, src_hbm, dst_hbm, zeros_hbm, out_hbm,
                     src_v, dst_v, rows_v, acc_sh, msg_sh, sem_a, sem_b):
        c = lax.axis_index("c")
        s = lax.axis_index("s")
        blk = c * NS + s
        rsl = pl.ds(s * (N // NS), N // NS)
        pltpu.sync_copy(zeros_hbm, acc_sh.at[rsl])
        pltpu.sync_copy(msg_hbm.at[rsl], msg_sh.at[rsl])
        pltpu.sync_copy(src_hbm.at[blk], src_v)
        pltpu.sync_copy(dst_hbm.at[blk], dst_v)
        plsc.subcore_barrier()
        sems = (sem_a, sem_b)
        handles = [None, None]
        handles[0] = pltpu.async_copy(msg_sh.at[src_v.at[0]], rows_v.at[0],
                                      sems[0])
        for j in range(NCHUNK):
            if j + 1 < NCHUNK:
                handles[(j + 1) % 2] = pltpu.async_copy(
                    msg_sh.at[src_v.at[j + 1]], rows_v.at[(j + 1) % 2],
                    sems[(j + 1) % 2])
            handles[j % 2].wait()
            pltpu.sync_copy(rows_v.at[j % 2], acc_sh.at[dst_v.at[j]], add=True)
        plsc.subcore_barrier()
        pltpu.sync_copy(acc_sh.at[pl.ds(s * (N // NS), N // NS)],
                        out_hbm.at[c].at[pl.ds(s * (N // NS), N // NS)])

    return sc_aggregate


TI = 512
TJ = 2048


def _dec_body(ab_ref, bv_ref, abt_ref, bvt_ref, w2_ref, bd2_ref, out_ref):
    a = ab_ref[...]
    b = bv_ref[...]
    at = abt_ref[...]
    bt = bvt_ref[...]
    zero = jnp.bfloat16(0.0)
    acc = jnp.zeros((TI, TJ), jnp.bfloat16)
    for k in range(L):
        wk = w2_ref[0, k]
        t = jnp.maximum(a[:, k:k + 1] + bt[k:k + 1, :], zero) \
            + jnp.maximum(b[:, k:k + 1] + at[k:k + 1, :], zero)
        acc += wk * t
    out_ref[...] = jax.nn.sigmoid(0.5 * acc.astype(F32) + bd2_ref[0, 0])


_decode = pl.pallas_call(
    _dec_body,
    grid=(N // TI, N // TJ),
    in_specs=[
        pl.BlockSpec((TI, L), lambda i, j: (i, 0)),
        pl.BlockSpec((TI, L), lambda i, j: (i, 0)),
        pl.BlockSpec((L, TJ), lambda i, j: (0, j)),
        pl.BlockSpec((L, TJ), lambda i, j: (0, j)),
        pl.BlockSpec(memory_space=pltpu.SMEM),
        pl.BlockSpec(memory_space=pltpu.SMEM),
    ],
    out_specs=pl.BlockSpec((TI, TJ), lambda i, j: (i, j)),
    out_shape=jax.ShapeDtypeStruct((N, N), F32),
)


def kernel(x, edge_index, Win, bin_, Wm0, bm0, wih0, whh0, bih0, bhh0,
           Wm1, bm1, wih1, whh1, bih1, bhh1,
           Wm2, bm2, wih2, whh2, bih2, bhh2,
           Wmu, bmu, Wls, bls, Wd1, bd1, Wd2, bd2):
    perm = lambda idx: (idx % PR) * P + idx // PR
    src = perm(edge_index[0]).reshape(NBLK, NCHUNK, CH)
    dst = perm(edge_index[1]).reshape(NBLK, NCHUNK, CH)
    zeros = jnp.zeros((N // NS, S), F32)
    w1a, w1b = Wd1[:L], Wd1[L:]
    rw = lambda v: v.reshape(1, -1)

    sc_aggregate = _make_sc_aggregate()
    state, msg = _enc0(x, Win, rw(bin_), Wm0, rw(bm0))
    agg = sc_aggregate(msg.reshape(N, S), src, dst, zeros)
    state, msg = _gru_step(state, agg.reshape(NC, PR, PW),
                           wih0, whh0, rw(bih0), rw(bhh0), Wm1, rw(bm1))
    agg = sc_aggregate(msg.reshape(N, S), src, dst, zeros)
    state, msg = _gru_step(state, agg.reshape(NC, PR, PW),
                           wih1, whh1, rw(bih1), rw(bhh1), Wm2, rw(bm2))
    agg = sc_aggregate(msg.reshape(N, S), src, dst, zeros)
    mu, logstd, ab, bv, abt, bvt = _final(
        state, agg.reshape(NC, PR, PW),
        wih2, whh2, rw(bih2), rw(bhh2), Wmu, rw(bmu), Wls, rw(bls),
        w1a, w1b, rw(bd1), bd1.reshape(L, 1))

    adj = _decode(ab, bv, abt, bvt,
                  Wd2.reshape(1, L).astype(jnp.bfloat16), bd2.reshape(1, 1))
    return (adj, mu, logstd)


# speedup vs baseline: 11.4511x; 1.0224x over previous
import functools

import jax
import jax.numpy as jnp
from jax import lax
from jax.experimental import pallas as pl
from jax.experimental.pallas import tpu as pltpu
from jax.experimental.pallas import tpu_sc as plsc

N = 2048
E = 65536
S = 32
L = 16

NC = 2
NS = 16
CH = 128
NBLK = NC * NS
NCHUNK = E // NBLK // CH

P = 4
PR = N // P
PW = P * S

F32 = jnp.float32


_DNT = (((1,), (1,)), ((), ()))


def _pmul(a, w, b, u_width=S):
    parts = [jnp.dot(a[:, u * u_width:(u + 1) * u_width], w,
                     preferred_element_type=F32) + b
             for u in range(P)]
    return jnp.concatenate(parts, axis=1)


def _enc0_body(x_ref, win_ref, bin_ref, wm_ref, bm_ref, state_ref, msg_ref):
    win, b = win_ref[...], bin_ref[...]
    parts = [jnp.maximum(
        jnp.dot(x_ref[u * PR:(u + 1) * PR, :], win,
                preferred_element_type=F32) + b, 0.0) for u in range(P)]
    st = jnp.concatenate(parts, axis=1)
    state_ref[...] = st
    msg_ref[...] = jnp.maximum(_pmul(st, wm_ref[...], bm_ref[...]), 0.0)


_enc0 = pl.pallas_call(
    _enc0_body,
    out_shape=(
        jax.ShapeDtypeStruct((PR, PW), F32),
        jax.ShapeDtypeStruct((PR, PW), F32),
    ),
)


def _half_gate(x, w, b):
    parts = [lax.dot_general(x[:, u * S:(u + 1) * S], w, _DNT,
                             preferred_element_type=F32) + b
             for u in range(P)]
    return jnp.concatenate(parts, axis=1)


def _gru_core(h, agg_ref, wih_ref, whh_ref, bih_ref, bhh_ref):
    a = agg_ref[0] + agg_ref[1]
    wih, whh = wih_ref[...], whh_ref[...]
    bih, bhh = bih_ref[...], bhh_ref[...]
    blk = lambda w, g: w[g * S:(g + 1) * S, :]
    bb = lambda b, g: b[:, g * S:(g + 1) * S]
    r = jax.nn.sigmoid(_half_gate(a, blk(wih, 0), bb(bih, 0))
                       + _half_gate(h, blk(whh, 0), bb(bhh, 0)))
    z = jax.nn.sigmoid(_half_gate(a, blk(wih, 1), bb(bih, 1))
                       + _half_gate(h, blk(whh, 1), bb(bhh, 1)))
    n = jnp.tanh(_half_gate(a, blk(wih, 2), bb(bih, 2))
                 + r * _half_gate(h, blk(whh, 2), bb(bhh, 2)))
    return h + (1.0 - z) * n + z * h


def _gru_body(state_ref, agg_ref, wih_ref, whh_ref, bih_ref, bhh_ref,
              wm_ref, bm_ref, newstate_ref, msg_ref):
    hn = _gru_core(state_ref[...], agg_ref, wih_ref, whh_ref,
                   bih_ref, bhh_ref)
    newstate_ref[...] = hn
    msg_ref[...] = jnp.maximum(_pmul(hn, wm_ref[...], bm_ref[...]), 0.0)


_gru_step = pl.pallas_call(
    _gru_body,
    out_shape=(
        jax.ShapeDtypeStruct((PR, PW), F32),
        jax.ShapeDtypeStruct((PR, PW), F32),
    ),
)


def _final_body(state_ref, agg_ref, wih_ref, whh_ref, bih_ref, bhh_ref,
                wmu_ref, bmu_ref, wls_ref, bls_ref,
                w1a_ref, w1b_ref, bd1_ref, bd1c_ref,
                mu_ref, ls_ref, ab_ref, bv_ref, abt_ref, bvt_ref):
    hn_p = _gru_core(state_ref[...], agg_ref, wih_ref, whh_ref,
                     bih_ref, bhh_ref)
    hn = jnp.concatenate([hn_p[:, u * S:(u + 1) * S] for u in range(P)], axis=0)
    mu = jnp.dot(hn, wmu_ref[...], preferred_element_type=F32) + bmu_ref[...]
    mu_ref[...] = mu
    ls_ref[...] = jnp.dot(hn, wls_ref[...], preferred_element_type=F32) + bls_ref[...]
    bf = jnp.bfloat16
    ab_ref[...] = (jnp.dot(mu, w1a_ref[...], preferred_element_type=F32)
                   + bd1_ref[...]).astype(bf)
    bv_ref[...] = jnp.dot(mu, w1b_ref[...],
                          preferred_element_type=F32).astype(bf)
    dn = (((0,), (1,)), ((), ()))
    abt_ref[...] = (lax.dot_general(w1a_ref[...], mu, dn,
                                    preferred_element_type=F32)
                    + bd1c_ref[...]).astype(bf)
    bvt_ref[...] = lax.dot_general(w1b_ref[...], mu, dn,
                                   preferred_element_type=F32).astype(bf)


_final = pl.pallas_call(
    _final_body,
    out_shape=(
        jax.ShapeDtypeStruct((N, L), F32),
        jax.ShapeDtypeStruct((N, L), F32),
        jax.ShapeDtypeStruct((N, L), jnp.bfloat16),
        jax.ShapeDtypeStruct((N, L), jnp.bfloat16),
        jax.ShapeDtypeStruct((L, N), jnp.bfloat16),
        jax.ShapeDtypeStruct((L, N), jnp.bfloat16),
    ),
)


@functools.cache
def _make_sc_aggregate():
    mesh = plsc.VectorSubcoreMesh(core_axis_name="c", subcore_axis_name="s")

    @functools.partial(
        pl.kernel,
        mesh=mesh,
        out_type=jax.ShapeDtypeStruct((NC, N, S), F32),
        scratch_types=[
            pltpu.VMEM((NCHUNK, CH), jnp.int32),
            pltpu.VMEM((NCHUNK, CH), jnp.int32),
            pltpu.VMEM((4, CH, S), F32),
            pltpu.VMEM_SHARED((N, S), F32),
            pltpu.VMEM_SHARED((N, S), F32),
            [pltpu.SemaphoreType.DMA] * 4,
            [pltpu.SemaphoreType.DMA] * 4,
        ],
        compiler_params=pltpu.CompilerParams(use_tc_tiling_on_sc=False),
    )
    def sc_aggregate(msg_hbm, src_hbm, dst_hbm, zeros_hbm, out_hbm,
                     src_v, dst_v, rows_v, acc_sh, msg_sh, gsems, ssems):
        c = lax.axis_index("c")
        s = lax.axis_index("s")
        blk = c * NS + s
        rsl = pl.ds(s * (N // NS), N // NS)
        pltpu.sync_copy(zeros_hbm, acc_sh.at[rsl])
        pltpu.sync_copy(msg_hbm.at[rsl], msg_sh.at[rsl])
        pltpu.sync_copy(src_hbm.at[blk], src_v)
        pltpu.sync_copy(dst_hbm.at[blk], dst_v)
        plsc.subcore_barrier()
        gh, sh = {}, {}

        def gather(j):
            return pltpu.async_copy(msg_sh.at[src_v.at[j]],
                                    rows_v.at[j % 4], gsems[j % 4])

        gh[0] = gather(0)
        gh[1] = gather(1)
        for j in range(NCHUNK):
            if j + 2 < NCHUNK:
                if j - 2 >= 0:
                    sh[j - 2].wait()
                gh[j + 2] = gather(j + 2)
            gh[j].wait()
            sh[j] = pltpu.async_copy(rows_v.at[j % 4],
                                     acc_sh.at[dst_v.at[j]],
                                     ssems[j % 4], add=True)
        for j in range(NCHUNK - 4, NCHUNK):
            sh[j].wait()
        plsc.subcore_barrier()
        pltpu.sync_copy(acc_sh.at[pl.ds(s * (N // NS), N // NS)],
                        out_hbm.at[c].at[pl.ds(s * (N // NS), N // NS)])

    return sc_aggregate


TI = 512
TJ = 2048


def _dec_body(ab_ref, bv_ref, abt_ref, bvt_ref, w2_ref, bd2_ref, out_ref):
    a = ab_ref[...]
    b = bv_ref[...]
    at = abt_ref[...]
    bt = bvt_ref[...]
    zero = jnp.bfloat16(0.0)
    acc = jnp.zeros((TI, TJ), jnp.bfloat16)
    for k in range(L):
        wk = w2_ref[0, k]
        t = jnp.maximum(a[:, k:k + 1] + bt[k:k + 1, :], zero) \
            + jnp.maximum(b[:, k:k + 1] + at[k:k + 1, :], zero)
        acc += wk * t
    out_ref[...] = jax.nn.sigmoid(0.5 * acc.astype(F32) + bd2_ref[0, 0])


_decode = pl.pallas_call(
    _dec_body,
    grid=(N // TI, N // TJ),
    in_specs=[
        pl.BlockSpec((TI, L), lambda i, j: (i, 0)),
        pl.BlockSpec((TI, L), lambda i, j: (i, 0)),
        pl.BlockSpec((L, TJ), lambda i, j: (0, j)),
        pl.BlockSpec((L, TJ), lambda i, j: (0, j)),
        pl.BlockSpec(memory_space=pltpu.SMEM),
        pl.BlockSpec(memory_space=pltpu.SMEM),
    ],
    out_specs=pl.BlockSpec((TI, TJ), lambda i, j: (i, j)),
    out_shape=jax.ShapeDtypeStruct((N, N), F32),
)


def kernel(x, edge_index, Win, bin_, Wm0, bm0, wih0, whh0, bih0, bhh0,
           Wm1, bm1, wih1, whh1, bih1, bhh1,
           Wm2, bm2, wih2, whh2, bih2, bhh2,
           Wmu, bmu, Wls, bls, Wd1, bd1, Wd2, bd2):
    perm = lambda idx: (idx % PR) * P + idx // PR
    src = perm(edge_index[0]).reshape(NBLK, NCHUNK, CH)
    dst = perm(edge_index[1]).reshape(NBLK, NCHUNK, CH)
    zeros = jnp.zeros((N // NS, S), F32)
    w1a, w1b = Wd1[:L], Wd1[L:]
    rw = lambda v: v.reshape(1, -1)

    sc_aggregate = _make_sc_aggregate()
    state, msg = _enc0(x, Win, rw(bin_), Wm0, rw(bm0))
    agg = sc_aggregate(msg.reshape(N, S), src, dst, zeros)
    state, msg = _gru_step(state, agg.reshape(NC, PR, PW),
                           wih0, whh0, rw(bih0), rw(bhh0), Wm1, rw(bm1))
    agg = sc_aggregate(msg.reshape(N, S), src, dst, zeros)
    state, msg = _gru_step(state, agg.reshape(NC, PR, PW),
                           wih1, whh1, rw(bih1), rw(bhh1), Wm2, rw(bm2))
    agg = sc_aggregate(msg.reshape(N, S), src, dst, zeros)
    mu, logstd, ab, bv, abt, bvt = _final(
        state, agg.reshape(NC, PR, PW),
        wih2, whh2, rw(bih2), rw(bhh2), Wmu, rw(bmu), Wls, rw(bls),
        w1a, w1b, rw(bd1), bd1.reshape(L, 1))

    adj = _decode(ab, bv, abt, bvt,
                  Wd2.reshape(1, L).astype(jnp.bfloat16), bd2.reshape(1, 1))
    return (adj, mu, logstd)
